# initial kernel scaffold (unmeasured)
import jax
import jax.numpy as jnp
from jax import lax
from jax.experimental import pallas as pl
from jax.experimental.pallas import tpu as pltpu

N_DEV = 32


def kernel(x, w_mat, scale_x, scale_w):
    m_per, k = x.shape
    _, n_per = w_mat.shape

    def body(x_ref, w_ref, sx_ref, sw_ref, out_ref, xg_ref, send_sems, recv_sems):
        my = lax.axis_index("i")
        left = lax.rem(my + N_DEV - 1, N_DEV)
        right = lax.rem(my + 1, N_DEV)

        barrier_sem = pltpu.get_barrier_semaphore()
        for nbr in (left, right):
            pl.semaphore_signal(
                barrier_sem, inc=1,
                device_id=(nbr,), device_id_type=pl.DeviceIdType.MESH,
            )
        pl.semaphore_wait(barrier_sem, 2)

        my_off = pl.multiple_of(my * m_per, m_per)
        xg_ref[pl.ds(my_off, m_per), :] = x_ref[:, :]

        for h in range(N_DEV - 1):
            send_origin = lax.rem(my - h + N_DEV, N_DEV)
            recv_origin = lax.rem(my - h - 1 + N_DEV, N_DEV)
            send_off = pl.multiple_of(send_origin * m_per, m_per)
            recv_off = pl.multiple_of(recv_origin * m_per, m_per)
            rdma = pltpu.make_async_remote_copy(
                src_ref=xg_ref.at[pl.ds(send_off, m_per), :],
                dst_ref=xg_ref.at[pl.ds(send_off, m_per), :],
                send_sem=send_sems.at[h],
                recv_sem=recv_sems.at[h],
                device_id=(right,),
                device_id_type=pl.DeviceIdType.MESH,
            )
            rdma.start()
            rdma.wait()
            del recv_off

        acc = jnp.dot(
            xg_ref[:, :], w_ref[:, :], preferred_element_type=jnp.float32
        )
        out_ref[:, :] = acc * (sx_ref[0] * sw_ref[0])

    return pl.pallas_call(
        body,
        out_shape=jax.ShapeDtypeStruct((N_DEV * m_per, n_per), jnp.float32),
        in_specs=[
            pl.BlockSpec(memory_space=pltpu.VMEM),
            pl.BlockSpec(memory_space=pltpu.VMEM),
            pl.BlockSpec(memory_space=pltpu.VMEM),
            pl.BlockSpec(memory_space=pltpu.VMEM),
        ],
        out_specs=pl.BlockSpec(memory_space=pltpu.VMEM),
        scratch_shapes=[
            pltpu.VMEM((N_DEV * m_per, k), x.dtype),
            pltpu.SemaphoreType.DMA((N_DEV - 1,)),
            pltpu.SemaphoreType.DMA((N_DEV - 1,)),
        ],
        compiler_params=pltpu.CompilerParams(collective_id=0),
    )(x, w_mat, scale_x, scale_w)


# baseline (device time: 248289 ns/iter reference)
import jax
import jax.numpy as jnp
from jax import lax
from jax.experimental import pallas as pl
from jax.experimental.pallas import tpu as pltpu

N_DEV = 32


def kernel(x, w_mat, scale_x, scale_w):
    m_per, k = x.shape
    _, n_per = w_mat.shape

    def body(x_ref, w_ref, sx_ref, sw_ref, out_ref, xg_ref, send_sems, recv_sems):
        my = lax.axis_index("i")
        left = lax.rem(my + N_DEV - 1, N_DEV)
        right = lax.rem(my + 1, N_DEV)

        barrier_sem = pltpu.get_barrier_semaphore()
        for nbr in (left, right):
            pl.semaphore_signal(
                barrier_sem, inc=1,
                device_id=(nbr,), device_id_type=pl.DeviceIdType.MESH,
            )
        pl.semaphore_wait(barrier_sem, 2)

        my_off = pl.multiple_of(my * m_per, m_per)
        xg_ref[pl.ds(my_off, m_per), :] = x_ref[:, :].astype(jnp.float8_e5m2)

        for h in range(N_DEV - 1):
            send_origin = lax.rem(my - h + N_DEV, N_DEV)
            recv_origin = lax.rem(my - h - 1 + N_DEV, N_DEV)
            send_off = pl.multiple_of(send_origin * m_per, m_per)
            recv_off = pl.multiple_of(recv_origin * m_per, m_per)
            rdma = pltpu.make_async_remote_copy(
                src_ref=xg_ref.at[pl.ds(send_off, m_per), :],
                dst_ref=xg_ref.at[pl.ds(send_off, m_per), :],
                send_sem=send_sems.at[h],
                recv_sem=recv_sems.at[h],
                device_id=(right,),
                device_id_type=pl.DeviceIdType.MESH,
            )
            rdma.start()
            rdma.wait()
            del recv_off

        w_bf = w_ref[:, :].astype(jnp.bfloat16)
        scale = sx_ref[0] * sw_ref[0]
        for s in range(N_DEV):
            rows = pl.ds(s * m_per, m_per)
            acc = jnp.dot(
                xg_ref[rows, :].astype(jnp.bfloat16), w_bf,
                preferred_element_type=jnp.float32,
            )
            out_ref[rows, :] = acc * scale

    return pl.pallas_call(
        body,
        out_shape=jax.ShapeDtypeStruct((N_DEV * m_per, n_per), jnp.float32),
        in_specs=[
            pl.BlockSpec(memory_space=pltpu.VMEM),
            pl.BlockSpec(memory_space=pltpu.VMEM),
            pl.BlockSpec(memory_space=pltpu.VMEM),
            pl.BlockSpec(memory_space=pltpu.VMEM),
        ],
        out_specs=pl.BlockSpec(memory_space=pltpu.VMEM),
        scratch_shapes=[
            pltpu.VMEM((N_DEV * m_per, k), jnp.float8_e5m2),
            pltpu.SemaphoreType.DMA((N_DEV - 1,)),
            pltpu.SemaphoreType.DMA((N_DEV - 1,)),
        ],
        compiler_params=pltpu.CompilerParams(collective_id=0),
    )(x, w_mat, scale_x, scale_w)


# device time: 193355 ns/iter; 1.2841x vs baseline; 1.2841x over previous
import jax
import jax.numpy as jnp
from jax import lax
from jax.experimental import pallas as pl
from jax.experimental.pallas import tpu as pltpu

N_DEV = 32
N_R = 16
N_L = 15


def kernel(x, w_mat, scale_x, scale_w):
    m_per, k = x.shape
    _, n_per = w_mat.shape

    def body(x_ref, w_ref, sx_ref, sw_ref, out_ref, xg_ref,
             r_send, r_recv, l_send, l_recv):
        my = lax.axis_index("i")
        left = lax.rem(my + N_DEV - 1, N_DEV)
        right = lax.rem(my + 1, N_DEV)

        barrier_sem = pltpu.get_barrier_semaphore()
        for nbr in (left, right):
            pl.semaphore_signal(
                barrier_sem, inc=1,
                device_id=(nbr,), device_id_type=pl.DeviceIdType.MESH,
            )
        pl.semaphore_wait(barrier_sem, 2)

        def origin_off(origin):
            return pl.multiple_of(origin * m_per, m_per)

        def rdma_to(nbr, off, send_sem, recv_sem):
            return pltpu.make_async_remote_copy(
                src_ref=xg_ref.at[pl.ds(off, m_per), :],
                dst_ref=xg_ref.at[pl.ds(off, m_per), :],
                send_sem=send_sem,
                recv_sem=recv_sem,
                device_id=(nbr,),
                device_id_type=pl.DeviceIdType.MESH,
            )

        my_off = origin_off(my)
        xg_ref[pl.ds(my_off, m_per), :] = x_ref[:, :].astype(jnp.float8_e5m2)

        rs = [rdma_to(right, my_off, r_send.at[0], r_recv.at[0])]
        ls = [rdma_to(left, my_off, l_send.at[0], l_recv.at[0])]
        rs[0].start()
        ls[0].start()

        w_bf = w_ref[:, :].astype(jnp.bfloat16)
        scale = sx_ref[0] * sw_ref[0]

        def gemm(off):
            rows = pl.ds(off, m_per)
            acc = jnp.dot(
                xg_ref[rows, :].astype(jnp.bfloat16), w_bf,
                preferred_element_type=jnp.float32,
            )
            out_ref[rows, :] = acc * scale

        gemm(my_off)

        for h in range(1, N_R + 1):
            off_r = origin_off(lax.rem(my - h + N_DEV, N_DEV))
            rs[h - 1].wait()
            if h < N_R:
                nxt = rdma_to(right, off_r, r_send.at[h], r_recv.at[h])
                nxt.start()
                rs.append(nxt)
            gemm(off_r)

            if h <= N_L:
                off_l = origin_off(lax.rem(my + h, N_DEV))
                ls[h - 1].wait()
                if h < N_L:
                    nxt = rdma_to(left, off_l, l_send.at[h], l_recv.at[h])
                    nxt.start()
                    ls.append(nxt)
                gemm(off_l)

    return pl.pallas_call(
        body,
        out_shape=jax.ShapeDtypeStruct((N_DEV * m_per, n_per), jnp.float32),
        in_specs=[
            pl.BlockSpec(memory_space=pltpu.VMEM),
            pl.BlockSpec(memory_space=pltpu.VMEM),
            pl.BlockSpec(memory_space=pltpu.VMEM),
            pl.BlockSpec(memory_space=pltpu.VMEM),
        ],
        out_specs=pl.BlockSpec(memory_space=pltpu.VMEM),
        scratch_shapes=[
            pltpu.VMEM((N_DEV * m_per, k), jnp.float8_e5m2),
            pltpu.SemaphoreType.DMA((N_R,)),
            pltpu.SemaphoreType.DMA((N_R,)),
            pltpu.SemaphoreType.DMA((N_L,)),
            pltpu.SemaphoreType.DMA((N_L,)),
        ],
        compiler_params=pltpu.CompilerParams(collective_id=0),
    )(x, w_mat, scale_x, scale_w)


# device time: 131531 ns/iter; 1.8877x vs baseline; 1.4700x over previous
import numpy as np

import jax
import jax.numpy as jnp
from jax import lax
from jax.experimental import pallas as pl
from jax.experimental.pallas import tpu as pltpu

N_DEV = 32
N_R = 16
N_L = 15


def _cycle_tables():
    def snake_id(x, y, z):
        return 8 * z + 2 * y + (x if y % 2 == 0 else 1 - x)

    path44 = []
    for z in range(4):
        ys = range(4) if z % 2 == 0 else reversed(range(4))
        path44.extend((y, z) for y in ys)
    cyc = [(0, y, z) for (y, z) in path44]
    cyc += [(1, y, z) for (y, z) in reversed(path44)]
    perm = np.array([snake_id(*c) for c in cyc])
    inv = np.empty(N_DEV, dtype=np.int64)
    inv[perm] = np.arange(N_DEV)

    right_tab = np.empty(N_DEV, dtype=np.int32)
    left_tab = np.empty(N_DEV, dtype=np.int32)
    r_orig_tab = np.empty((N_DEV, N_R), dtype=np.int32)
    l_orig_tab = np.empty((N_DEV, N_L), dtype=np.int32)
    for m in range(N_DEV):
        p = inv[m]
        right_tab[m] = perm[(p + 1) % N_DEV]
        left_tab[m] = perm[(p - 1) % N_DEV]
        for h in range(1, N_R + 1):
            r_orig_tab[m, h - 1] = perm[(p - h) % N_DEV]
        for h in range(1, N_L + 1):
            l_orig_tab[m, h - 1] = perm[(p + h) % N_DEV]
    return right_tab, left_tab, r_orig_tab, l_orig_tab


_RIGHT_TAB, _LEFT_TAB, _R_ORIG_TAB, _L_ORIG_TAB = _cycle_tables()


def kernel(x, w_mat, scale_x, scale_w):
    m_per, k = x.shape
    _, n_per = w_mat.shape

    my = lax.axis_index("i")
    nbrs = jnp.stack(
        [jnp.asarray(_LEFT_TAB)[my], jnp.asarray(_RIGHT_TAB)[my]]
    ).astype(jnp.int32)
    r_orig = jnp.asarray(_R_ORIG_TAB)[my]
    l_orig = jnp.asarray(_L_ORIG_TAB)[my]

    def body(nbrs_ref, r_orig_ref, l_orig_ref,
             x_ref, w_ref, sx_ref, sw_ref, out_ref, xg_ref,
             r_send, r_recv, l_send, l_recv):
        my_id = lax.axis_index("i")
        left = nbrs_ref[0]
        right = nbrs_ref[1]

        barrier_sem = pltpu.get_barrier_semaphore()
        for nbr in (left, right):
            pl.semaphore_signal(
                barrier_sem, inc=1,
                device_id=(nbr,), device_id_type=pl.DeviceIdType.MESH,
            )
        pl.semaphore_wait(barrier_sem, 2)

        def rdma_to(nbr, off, send_sem, recv_sem):
            return pltpu.make_async_remote_copy(
                src_ref=xg_ref.at[pl.ds(off, m_per), :],
                dst_ref=xg_ref.at[pl.ds(off, m_per), :],
                send_sem=send_sem,
                recv_sem=recv_sem,
                device_id=(nbr,),
                device_id_type=pl.DeviceIdType.MESH,
            )

        my_off = pl.multiple_of(my_id * m_per, m_per)
        xg_ref[pl.ds(my_off, m_per), :] = x_ref[:, :].astype(jnp.float8_e5m2)

        rs = [rdma_to(right, my_off, r_send.at[0], r_recv.at[0])]
        ls = [rdma_to(left, my_off, l_send.at[0], l_recv.at[0])]
        rs[0].start()
        ls[0].start()

        w_bf = w_ref[:, :].astype(jnp.bfloat16)
        scale = sx_ref[0] * sw_ref[0]

        def gemm(off):
            rows = pl.ds(off, m_per)
            acc = jnp.dot(
                xg_ref[rows, :].astype(jnp.bfloat16), w_bf,
                preferred_element_type=jnp.float32,
            )
            out_ref[rows, :] = acc * scale

        gemm(my_off)

        for h in range(1, N_R + 1):
            off_r = pl.multiple_of(r_orig_ref[h - 1] * m_per, m_per)
            rs[h - 1].wait()
            if h < N_R:
                nxt = rdma_to(right, off_r, r_send.at[h], r_recv.at[h])
                nxt.start()
                rs.append(nxt)
            gemm(off_r)

            if h <= N_L:
                off_l = pl.multiple_of(l_orig_ref[h - 1] * m_per, m_per)
                ls[h - 1].wait()
                if h < N_L:
                    nxt = rdma_to(left, off_l, l_send.at[h], l_recv.at[h])
                    nxt.start()
                    ls.append(nxt)
                gemm(off_l)

    return pl.pallas_call(
        body,
        out_shape=jax.ShapeDtypeStruct((N_DEV * m_per, n_per), jnp.float32),
        in_specs=[
            pl.BlockSpec(memory_space=pltpu.SMEM),
            pl.BlockSpec(memory_space=pltpu.SMEM),
            pl.BlockSpec(memory_space=pltpu.SMEM),
            pl.BlockSpec(memory_space=pltpu.VMEM),
            pl.BlockSpec(memory_space=pltpu.VMEM),
            pl.BlockSpec(memory_space=pltpu.VMEM),
            pl.BlockSpec(memory_space=pltpu.VMEM),
        ],
        out_specs=pl.BlockSpec(memory_space=pltpu.VMEM),
        scratch_shapes=[
            pltpu.VMEM((N_DEV * m_per, k), jnp.float8_e5m2),
            pltpu.SemaphoreType.DMA((N_R,)),
            pltpu.SemaphoreType.DMA((N_R,)),
            pltpu.SemaphoreType.DMA((N_L,)),
            pltpu.SemaphoreType.DMA((N_L,)),
        ],
        compiler_params=pltpu.CompilerParams(collective_id=0),
    )(nbrs, r_orig, l_orig, x, w_mat, scale_x, scale_w)
